# Initial kernel scaffold; baseline (speedup 1.0000x reference)
import jax
import jax.numpy as jnp
from jax.experimental import pallas as pl


def kernel(x_degree, x_pos, edge_index, dense_ori, dense_spd, dense_index, params):
    raise NotImplementedError("write your pallas kernel here")


# speedup vs baseline: 26.0431x; 26.0431x over previous
import functools

import jax
import jax.numpy as jnp
from jax import lax
from jax.experimental import pallas as pl
from jax.experimental.pallas import tpu as pltpu
from jax.experimental.pallas import tpu_sc as plsc

B = 8
N = 128
BN = B * N
E = 32768
EB = E // B
ED = 32
OUT = 64
HEADS = 4
POS = 16
ISQ2 = 0.7071067811865476

_F = jnp.float32


def _silu(x):
    return x / (1.0 + jnp.exp(-x))


def _dot(a, b):
    return jnp.dot(a, b, preferred_element_type=_F)


def _gat(ohT, tab):
    return lax.dot_general(ohT, tab, (((0,), (0,)), ((), ())),
                           preferred_element_type=_F)


def _sc_gather(ori2, spd2, idx):
    info = plsc.get_sparse_core_info()
    nw = info.num_cores * info.num_subcores
    bpw = E // nw
    ch = bpw // 128
    idx3 = idx.reshape(nw, ch, 128)
    mesh = plsc.VectorSubcoreMesh(core_axis_name="c", subcore_axis_name="s")

    @functools.partial(
        pl.kernel,
        mesh=mesh,
        out_type=(jax.ShapeDtypeStruct((E, ED), _F),
                  jax.ShapeDtypeStruct((E, ED), _F)),
        scratch_types=[
            pltpu.VMEM((ch, 128), jnp.int32),
            pltpu.VMEM((bpw, ED), _F),
            pltpu.VMEM((bpw, ED), _F),
            pltpu.SemaphoreType.DMA,
        ],
        compiler_params=pltpu.CompilerParams(use_tc_tiling_on_sc=False),
    )
    def gather(ori_hbm, spd_hbm, idx_hbm, go_hbm, gs_hbm, idx_v, r1, r2, sem):
        wid = lax.axis_index("s") * info.num_cores + lax.axis_index("c")
        base = wid * bpw
        pltpu.sync_copy(idx_hbm.at[wid], idx_v)
        cps = []
        for j in range(ch):
            cps.append(pltpu.async_copy(
                ori_hbm.at[idx_v.at[j]], r1.at[pl.ds(j * 128, 128)], sem))
            cps.append(pltpu.async_copy(
                spd_hbm.at[idx_v.at[j]], r2.at[pl.ds(j * 128, 128)], sem))
        for c in cps:
            c.wait()
        pltpu.sync_copy(r1, go_hbm.at[pl.ds(base, bpw)])
        pltpu.sync_copy(r2, gs_hbm.at[pl.ds(base, bpw)])

    return gather(ori2, spd2, idx3)


def _conv_body(xd_ref, xp_ref, go_ref, gs_ref, src_ref, dst_ref, *rest):
    wrefs = rest[:33]
    qouts = rest[33:]

    h = xd_ref[0]
    hp = xp_ref[0]
    g = jnp.concatenate([go_ref[0], gs_ref[0]], axis=1)
    src = src_ref[0]
    dst = dst_ref[0]

    iota_n = lax.broadcasted_iota(jnp.int32, (N, EB), 0)
    ohs = (jnp.broadcast_to(src, (N, EB)) == iota_n).astype(_F)
    ohd = (jnp.broadcast_to(dst, (N, EB)) == iota_n).astype(_F)
    ohsum = ohs + ohd
    cnt = jnp.maximum(jnp.sum(ohd, axis=1, keepdims=True), 1.0)

    hm = (lax.broadcasted_iota(jnp.int32, (OUT, HEADS), 0) // 16
          == lax.broadcasted_iota(jnp.int32, (OUT, HEADS), 1)).astype(_F)
    hmT = (lax.broadcasted_iota(jnp.int32, (HEADS, OUT), 0)
           == lax.broadcasted_iota(jnp.int32, (HEADS, OUT), 1) // 16).astype(_F)

    qtabs = []
    for i in range(3):
        (wq, bq, wk, bk, wv, bv, we, be, wpos, wec, bec) = (
            r[...] for r in wrefs[11 * i:11 * (i + 1)])
        qn = _dot(h, wq) + bq
        kn = _dot(h, wk) + bk
        vn = _dot(h, wv) + bv
        pmn = _dot(hp, wpos)

        gq = _gat(ohd, qn)
        gk = _gat(ohs, kn)
        gv = _gat(ohs, vn)
        gpm = _gat(ohs, pmn)

        if i == 0:
            he = g
        else:
            he = g * (ISQ2 ** i)
            for j in range(i):
                becj = wrefs[11 * j + 10][...]
                qsum = _gat(ohsum, qtabs[j])
                he = he + (ISQ2 ** (i - j)) * _silu(qsum + becj)

        ek = _dot(he, we) + be
        k_e = gk + ek
        v_e = gv + ek
        logits = _dot(gq * k_e, hm) * 0.25

        s = _dot(ohd, logits) / cnt
        ex = jnp.exp(logits - _gat(ohd, s))
        den = _dot(ohd, ex)
        alpha = ex / (_gat(ohd, den) + 1e-16)

        aexp = _dot(alpha, hmT)
        h = _dot(ohd, aexp * v_e)
        posw = (jnp.sum(alpha, axis=1, keepdims=True) * 0.25) * gpm
        hp = jnp.tanh(hp + _dot(ohd, posw))

        qi = _dot(h, wec)
        qtabs.append(qi)
        qouts[i][0] = qi


_RT = 32


def _final_body(ori_ref, spd_ref, q0r, q0c, q1r, q1c, q2r, q2c,
                wel_ref, bel_ref, b0_ref, b1_ref, b2_ref, out_ref):
    wel = wel_ref[...]
    w0 = wel[0:OUT] + wel[OUT:2 * OUT] * (ISQ2 * 0.5)
    fo = ori_ref[0].reshape(_RT * N, ED)
    fs = spd_ref[0].reshape(_RT * N, ED)
    acc = _dot(fo, w0[0:ED]) + _dot(fs, w0[ED:2 * ED]) + bel_ref[...]

    scales = (ISQ2 * 0.5, 0.5, ISQ2)
    for (qr, qc, br, sc) in ((q0r, q0c, b0_ref, scales[0]),
                             (q1r, q1c, b1_ref, scales[1]),
                             (q2r, q2c, b2_ref, scales[2])):
        qrow = jnp.broadcast_to(qr[0][:, None, :], (_RT, N, OUT))
        qcol = jnp.broadcast_to(qc[0][None, :, :], (_RT, N, OUT))
        s_t = _silu((qrow + qcol).reshape(_RT * N, OUT) + br[...])
        acc = acc + _dot(s_t, wel[OUT:2 * OUT]) * sc

    out_ref[0] = acc.reshape(_RT, N, ED)


def kernel(x_degree, x_pos, edge_index, dense_ori, dense_spd, dense_index,
           params):
    ei = edge_index.astype(jnp.int32)
    di = dense_index.astype(jnp.int32)
    ori2 = dense_ori.reshape(BN * N, ED)
    spd2 = dense_spd.reshape(BN * N, ED)

    g_ori, g_spd = _sc_gather(ori2, spd2, di)

    offs = (jnp.arange(B, dtype=jnp.int32) * N)[:, None]
    srcl = (ei[0].reshape(B, EB) - offs).reshape(B, 1, EB)
    dstl = (ei[1].reshape(B, EB) - offs).reshape(B, 1, EB)

    xd3 = x_degree.reshape(B, N, x_degree.shape[1])
    xp3 = x_pos.reshape(B, N, POS)
    go3 = g_ori.reshape(B, EB, ED)
    gs3 = g_spd.reshape(B, EB, ED)

    wlist = []
    for i in range(3):
        p = params["convs"][i]
        wec, bec = params["edge_convs"][i]
        wlist += [p["Wq"], p["bq"].reshape(1, OUT), p["Wk"],
                  p["bk"].reshape(1, OUT), p["Wv"], p["bv"].reshape(1, OUT),
                  p["We"], p["be"].reshape(1, OUT), p["Wpos"], wec,
                  bec.reshape(1, OUT)]

    def _full(a):
        nd = a.ndim
        return pl.BlockSpec(a.shape, lambda b, _n=nd: (0,) * _n)

    in_specs = [
        pl.BlockSpec((1, N, x_degree.shape[1]), lambda b: (b, 0, 0)),
        pl.BlockSpec((1, N, POS), lambda b: (b, 0, 0)),
        pl.BlockSpec((1, EB, ED), lambda b: (b, 0, 0)),
        pl.BlockSpec((1, EB, ED), lambda b: (b, 0, 0)),
        pl.BlockSpec((1, 1, EB), lambda b: (b, 0, 0)),
        pl.BlockSpec((1, 1, EB), lambda b: (b, 0, 0)),
    ] + [_full(a) for a in wlist]

    q0, q1, q2 = pl.pallas_call(
        _conv_body,
        grid=(B,),
        in_specs=in_specs,
        out_specs=[pl.BlockSpec((1, N, OUT), lambda b: (b, 0, 0))] * 3,
        out_shape=[jax.ShapeDtypeStruct((B, N, OUT), _F)] * 3,
    )(xd3, xp3, go3, gs3, srcl, dstl, *wlist)

    wel, bel = params["edge_layer"]
    becs = [params["edge_convs"][i][1].reshape(1, OUT) for i in range(3)]

    nrt = N // _RT
    qrow_spec = pl.BlockSpec((1, _RT, OUT), lambda b, r: (b, r, 0))
    qcol_spec = pl.BlockSpec((1, N, OUT), lambda b, r: (b, 0, 0))

    def _full2(a):
        nd = a.ndim
        return pl.BlockSpec(a.shape, lambda b, r, _n=nd: (0,) * _n)

    out4 = pl.pallas_call(
        _final_body,
        grid=(B, nrt),
        in_specs=[
            pl.BlockSpec((1, _RT, N, ED), lambda b, r: (b, r, 0, 0)),
            pl.BlockSpec((1, _RT, N, ED), lambda b, r: (b, r, 0, 0)),
            qrow_spec, qcol_spec, qrow_spec, qcol_spec, qrow_spec, qcol_spec,
            _full2(wel), _full2(bel.reshape(1, ED)),
            _full2(becs[0]), _full2(becs[1]), _full2(becs[2]),
        ],
        out_specs=pl.BlockSpec((1, _RT, N, ED), lambda b, r: (b, r, 0, 0)),
        out_shape=jax.ShapeDtypeStruct((B, N, N, ED), _F),
    )(dense_ori, dense_spd, q0, q0, q1, q1, q2, q2,
      wel, bel.reshape(1, ED), *becs)

    return out4.transpose(0, 3, 1, 2)
